# baseline (speedup 1.0000x reference)
import functools

import jax
import jax.numpy as jnp
from jax import lax
from jax.experimental import pallas as pl
from jax.experimental.pallas import tpu as pltpu
from jax.experimental.pallas import tpu_sc as plsc

_H = 4096
_N = 4 * 4096
_NC, _NS = 2, 16
_NW = _NC * _NS
_C = 16
_NCH = 34
_SPW = _C * _NCH
_S = _NW * _SPW
_K = 4
_NCHP = 40


def _sc_scatter(srcs, dest_idx):
    mesh = plsc.VectorSubcoreMesh(core_axis_name="c", subcore_axis_name="s")

    @functools.partial(
        pl.kernel,
        mesh=mesh,
        out_type=jax.ShapeDtypeStruct((_N, _H), jnp.float32),
        scratch_types=[
            pltpu.VMEM((_C, _H), jnp.float32),
            pltpu.VMEM((_NCHP, _C), jnp.int32),
            pltpu.SemaphoreType.DMA,
            pltpu.SemaphoreType.DMA,
        ],
    )
    def k(src_hbm, dest_hbm, out_hbm, src_v, idx_v, lsem, wsem):
        wid = lax.axis_index("s") * _NC + lax.axis_index("c")
        pltpu.async_copy(src_hbm.at[pl.ds(wid * _C, _C)], src_v, lsem).wait()
        pltpu.async_copy(
            dest_hbm.at[pl.ds(wid * _NCHP, _NCHP)], idx_v, lsem
        ).wait()

        def scatter(c):
            pltpu.async_copy(
                src_v, out_hbm.at[idx_v.at[c, pl.ds(0, _C)]], wsem
            )

        def drain(c):
            pltpu.make_async_copy(
                src_v, out_hbm.at[idx_v.at[c, pl.ds(0, _C)]], wsem
            ).wait()

        @pl.loop(0, _NCH)
        def _(c):
            @pl.when(c >= _K)
            def _():
                drain(c)

            scatter(c)

        @pl.loop(0, _K)
        def _(c):
            drain(c)

    return k(srcs, dest_idx)


def kernel(token_type_ids, token_type_embeddings):
    flat = token_type_ids.reshape(_N)
    c0 = jnp.sum(flat == 0).astype(jnp.int32)
    c1 = (_N - c0).astype(jnp.int32)
    order = jnp.argsort(flat, stable=True).astype(jnp.int32)
    zr = _SPW * ((c0 + _SPW - 1) // _SPW)
    i = jnp.arange(_S, dtype=jnp.int32)
    ones_exist = c1 > 0
    in_ones = (i >= zr) & ones_exist
    dest = jnp.where(
        in_ones,
        order[c0 + jnp.clip(i - zr, 0, c1 - 1)],
        order[jnp.clip(i, 0, c0 - 1)],
    )
    dest_idx = jnp.pad(
        dest.reshape(_NW, _NCH, _C), ((0, 0), (0, _NCHP - _NCH), (0, 0))
    ).reshape(_NW * _NCHP, _C)
    b_w = ((jnp.arange(_NW, dtype=jnp.int32) * _SPW >= zr) & ones_exist)
    rows = jnp.take(token_type_embeddings, b_w.astype(jnp.int32), axis=0)
    srcs = jnp.broadcast_to(rows[:, None, :], (_NW, _C, _H)).reshape(
        _NW * _C, _H
    )
    out = _sc_scatter(srcs, dest_idx)
    return out.reshape(token_type_ids.shape + (_H,))


# speedup vs baseline: 1.5289x; 1.5289x over previous
import functools

import jax
import jax.numpy as jnp
from jax import lax
from jax.experimental import pallas as pl
from jax.experimental.pallas import tpu as pltpu
from jax.experimental.pallas import tpu_sc as plsc

_H = 4096
_N = 4 * 4096
_NC, _NS = 2, 16
_NW = _NC * _NS
_C = 16
_NCH = 34
_SPW = _C * _NCH
_S = _NW * _SPW
_K = 4
_NCHP = 40


def _sc_scatter(srcs, dest_idx):
    mesh = plsc.VectorSubcoreMesh(core_axis_name="c", subcore_axis_name="s")

    @functools.partial(
        pl.kernel,
        mesh=mesh,
        out_type=jax.ShapeDtypeStruct((_N, _H), jnp.float32),
        scratch_types=[
            pltpu.VMEM((_C, _H), jnp.float32),
            pltpu.VMEM((_NCHP, _C), jnp.int32),
            pltpu.SemaphoreType.DMA,
            pltpu.SemaphoreType.DMA,
        ],
    )
    def k(src_hbm, dest_hbm, out_hbm, src_v, idx_v, lsem, wsem):
        wid = lax.axis_index("s") * _NC + lax.axis_index("c")
        pltpu.async_copy(src_hbm.at[pl.ds(wid * _C, _C)], src_v, lsem).wait()
        pltpu.async_copy(
            dest_hbm.at[pl.ds(wid * _NCHP, _NCHP)], idx_v, lsem
        ).wait()

        def scatter(c):
            pltpu.async_copy(
                src_v, out_hbm.at[idx_v.at[c, pl.ds(0, _C)]], wsem
            )

        def drain(c):
            pltpu.make_async_copy(
                src_v, out_hbm.at[idx_v.at[c, pl.ds(0, _C)]], wsem
            ).wait()

        @pl.loop(0, _NCH)
        def _(c):
            @pl.when(c >= _K)
            def _():
                drain(c)

            scatter(c)

        @pl.loop(0, _K)
        def _(c):
            drain(c)

    return k(srcs, dest_idx)


def kernel(token_type_ids, token_type_embeddings):
    flat = token_type_ids.reshape(_N)
    c0 = jnp.sum(flat == 0).astype(jnp.int32)
    c1 = (_N - c0).astype(jnp.int32)
    order = jnp.argsort(flat, stable=True).astype(jnp.int32)
    zr = _SPW * ((c0 + _SPW - 1) // _SPW)
    i = jnp.arange(_S, dtype=jnp.int32)
    ones_exist = c1 > 0
    in_ones = (i >= zr) & ones_exist
    rolled = jnp.roll(order, zr - c0)
    ordered_pad = jnp.concatenate([order, jnp.zeros((_S - _N,), jnp.int32)])
    rolled_pad = jnp.concatenate([rolled, rolled[: _S - _N]])
    pad0 = lax.dynamic_index_in_dim(order, jnp.maximum(c0 - 1, 0), keepdims=False)
    pad1 = order[_N - 1]
    dest = jnp.where(
        in_ones,
        jnp.where(i < zr + c1, rolled_pad, pad1),
        jnp.where(i < c0, ordered_pad, pad0),
    )
    dest_idx = jnp.pad(
        dest.reshape(_NW, _NCH, _C), ((0, 0), (0, _NCHP - _NCH), (0, 0))
    ).reshape(_NW * _NCHP, _C)
    b_w = ((jnp.arange(_NW, dtype=jnp.int32) * _SPW >= zr) & ones_exist)
    sel = b_w.astype(jnp.float32)[:, None]
    rows = (
        token_type_embeddings[0][None, :] * (1.0 - sel)
        + token_type_embeddings[1][None, :] * sel
    )
    srcs = jnp.broadcast_to(rows[:, None, :], (_NW, _C, _H)).reshape(
        _NW * _C, _H
    )
    out = _sc_scatter(srcs, dest_idx)
    return out.reshape(token_type_ids.shape + (_H,))


# speedup vs baseline: 1.5289x; 1.0000x over previous
import functools

import jax
import jax.numpy as jnp
from jax import lax
from jax.experimental import pallas as pl
from jax.experimental.pallas import tpu as pltpu
from jax.experimental.pallas import tpu_sc as plsc

_H = 4096
_N = 4 * 4096
_NC, _NS = 2, 16
_NW = _NC * _NS
_C = 16
_NCH = 34
_SPW = _C * _NCH
_S = _NW * _SPW
_K = 4
_NCHP = 40


def _sc_scatter(srcs, dest_idx):
    mesh = plsc.VectorSubcoreMesh(core_axis_name="c", subcore_axis_name="s")

    @functools.partial(
        pl.kernel,
        mesh=mesh,
        out_type=jax.ShapeDtypeStruct((_N, _H), jnp.float32),
        scratch_types=[
            pltpu.VMEM((_C, _H), jnp.float32),
            pltpu.VMEM((_NCHP, _C), jnp.int32),
            pltpu.SemaphoreType.DMA,
            pltpu.SemaphoreType.DMA,
        ],
    )
    def k(src_hbm, dest_hbm, out_hbm, src_v, idx_v, lsem, wsem):
        wid = lax.axis_index("s") * _NC + lax.axis_index("c")
        pltpu.async_copy(src_hbm.at[pl.ds(wid * _C, _C)], src_v, lsem).wait()
        pltpu.async_copy(
            dest_hbm.at[pl.ds(wid * _NCHP, _NCHP)], idx_v, lsem
        ).wait()

        def scatter(c):
            pltpu.async_copy(
                src_v, out_hbm.at[idx_v.at[c, pl.ds(0, _C)]], wsem
            )

        def drain(c):
            pltpu.make_async_copy(
                src_v, out_hbm.at[idx_v.at[c, pl.ds(0, _C)]], wsem
            ).wait()

        @pl.loop(0, _NCH)
        def _(c):
            scatter(c)

        @pl.loop(0, _NCH)
        def _(c):
            drain(c)

    return k(srcs, dest_idx)


def kernel(token_type_ids, token_type_embeddings):
    flat = token_type_ids.reshape(_N)
    c0 = jnp.sum(flat == 0).astype(jnp.int32)
    c1 = (_N - c0).astype(jnp.int32)
    order = jnp.argsort(flat, stable=True).astype(jnp.int32)
    zr = _SPW * ((c0 + _SPW - 1) // _SPW)
    i = jnp.arange(_S, dtype=jnp.int32)
    ones_exist = c1 > 0
    in_ones = (i >= zr) & ones_exist
    rolled = jnp.roll(order, zr - c0)
    ordered_pad = jnp.concatenate([order, jnp.zeros((_S - _N,), jnp.int32)])
    rolled_pad = jnp.concatenate([rolled, rolled[: _S - _N]])
    pad0 = lax.dynamic_index_in_dim(order, jnp.maximum(c0 - 1, 0), keepdims=False)
    pad1 = order[_N - 1]
    dest = jnp.where(
        in_ones,
        jnp.where(i < zr + c1, rolled_pad, pad1),
        jnp.where(i < c0, ordered_pad, pad0),
    )
    dest_idx = jnp.pad(
        dest.reshape(_NW, _NCH, _C), ((0, 0), (0, _NCHP - _NCH), (0, 0))
    ).reshape(_NW * _NCHP, _C)
    b_w = ((jnp.arange(_NW, dtype=jnp.int32) * _SPW >= zr) & ones_exist)
    sel = b_w.astype(jnp.float32)[:, None]
    rows = (
        token_type_embeddings[0][None, :] * (1.0 - sel)
        + token_type_embeddings[1][None, :] * sel
    )
    srcs = jnp.broadcast_to(rows[:, None, :], (_NW, _C, _H)).reshape(
        _NW * _C, _H
    )
    out = _sc_scatter(srcs, dest_idx)
    return out.reshape(token_type_ids.shape + (_H,))
